# baseline (speedup 1.0000x reference)
import jax
import jax.numpy as jnp
from jax.experimental import pallas as pl
from jax.experimental.pallas import tpu as pltpu

_SAMPLE_N = 16384
_FEAT = 128
_BLOCK = 2048


def _inner(x_ref, o_ref):
    o_ref[...] = x_ref[...]


def _outer(x_hbm, o_hbm):
    pipeline = pltpu.emit_pipeline(
        _inner,
        grid=(_SAMPLE_N // _BLOCK,),
        in_specs=[pl.BlockSpec((_BLOCK, _FEAT), lambda i: (i, 0))],
        out_specs=[pl.BlockSpec((_BLOCK, _FEAT), lambda i: (i, 0))],
    )
    pipeline(x_hbm, o_hbm)


def kernel(dataset):
    return pl.pallas_call(
        _outer,
        in_specs=[pl.BlockSpec(memory_space=pltpu.MemorySpace.HBM)],
        out_specs=pl.BlockSpec(memory_space=pltpu.MemorySpace.HBM),
        out_shape=jax.ShapeDtypeStruct((_SAMPLE_N, _FEAT), jnp.float32),
    )(dataset)
